# baseline (speedup 1.0000x reference)
import functools

import jax
import jax.numpy as jnp
from jax import lax
from jax.experimental import pallas as pl
from jax.experimental.pallas import tpu as pltpu
from jax.experimental.pallas import tpu_sc as plsc

EMB_DIM = 32
BATCH = 16384
NC, NS = 2, 16
NW = NC * NS
BPW = BATCH // NW


def _sc_gather(idx0, idx1, table0, table1):
    mesh = plsc.VectorSubcoreMesh(core_axis_name="c", subcore_axis_name="s")

    @functools.partial(
        pl.kernel,
        out_type=[
            jax.ShapeDtypeStruct((BATCH, EMB_DIM), jnp.float32),
            jax.ShapeDtypeStruct((BATCH, EMB_DIM), jnp.float32),
        ],
        mesh=mesh,
        scratch_types=[
            pltpu.VMEM((BPW,), jnp.int32),
            pltpu.VMEM((BPW,), jnp.int32),
            pltpu.VMEM((BPW, EMB_DIM), jnp.float32),
            pltpu.VMEM((BPW, EMB_DIM), jnp.float32),
            pltpu.SemaphoreType.DMA,
            pltpu.SemaphoreType.DMA,
        ],
        compiler_params=pltpu.CompilerParams(use_tc_tiling_on_sc=False),
    )
    def gather_k(idx0_hbm, idx1_hbm, t0_hbm, t1_hbm, e0_hbm, e1_hbm,
                 idx0_v, idx1_v, rows0_v, rows1_v, sem0, sem1):
        wid = lax.axis_index("s") * NC + lax.axis_index("c")
        base = wid * BPW
        pltpu.sync_copy(idx0_hbm.at[pl.ds(base, BPW)], idx0_v)
        pltpu.sync_copy(idx1_hbm.at[pl.ds(base, BPW)], idx1_v)
        c0 = pltpu.async_copy(t0_hbm.at[idx0_v], rows0_v, sem0)
        c1 = pltpu.async_copy(t1_hbm.at[idx1_v], rows1_v, sem1)
        c0.wait()
        c1.wait()
        pltpu.sync_copy(rows0_v, e0_hbm.at[pl.ds(base, BPW)])
        pltpu.sync_copy(rows1_v, e1_hbm.at[pl.ds(base, BPW)])

    return gather_k(idx0, idx1, table0, table1)


def _tc_mlp(e0, e1, w1a, w1b, b1, w2, b2):
    bs = 2048
    grid = (BATCH // bs,)

    def mlp_body(e0_ref, e1_ref, w1a_ref, w1b_ref, b1_ref, w2_ref, b2_ref,
                 out_ref):
        z = (e0_ref[...] @ w1a_ref[...] + e1_ref[...] @ w1b_ref[...]
             + b1_ref[...])
        a = jnp.maximum(z, 0.0)
        out_ref[...] = a @ w2_ref[...] + b2_ref[...]

    small = lambda shape: pl.BlockSpec(shape, lambda i: (0,) * len(shape))
    return pl.pallas_call(
        mlp_body,
        grid=grid,
        in_specs=[
            pl.BlockSpec((bs, EMB_DIM), lambda i: (i, 0)),
            pl.BlockSpec((bs, EMB_DIM), lambda i: (i, 0)),
            small((EMB_DIM, 4)),
            small((EMB_DIM, 4)),
            small((1, 4)),
            small((4, 1)),
            small((1, 1)),
        ],
        out_specs=pl.BlockSpec((bs, 1), lambda i: (i, 0)),
        out_shape=jax.ShapeDtypeStruct((BATCH, 1), jnp.float32),
    )(e0, e1, w1a, w1b, b1, w2, b2)


def kernel(indices, table0, table1, W1, b1, W2, b2):
    idx0 = indices[0].astype(jnp.int32)
    idx1 = indices[1].astype(jnp.int32)
    e0, e1 = _sc_gather(idx0, idx1, table0, table1)
    w1a = W1[:, :EMB_DIM].T
    w1b = W1[:, EMB_DIM:].T
    return _tc_mlp(e0, e1, w1a, w1b, b1.reshape(1, 4), W2.T,
                   b2.reshape(1, 1))


# speedup vs baseline: 1.4739x; 1.4739x over previous
import functools

import jax
import jax.numpy as jnp
from jax import lax
from jax.experimental import pallas as pl
from jax.experimental.pallas import tpu as pltpu
from jax.experimental.pallas import tpu_sc as plsc

EMB_DIM = 32
BATCH = 16384
NC, NS, L = 2, 16, 16
NW = NC * NS
BPW = BATCH // NW
CHUNK = 64
NCHUNK = BPW // CHUNK
NPARAM = 272


def _splat(ref, i):
    blk = ref[pl.ds((i // L) * L, L)]
    return lax.broadcast(blk[i % L], (L,))


def _fused_dlrm(idx0, idx1, table0, table1, params):
    mesh = plsc.VectorSubcoreMesh(core_axis_name="c", subcore_axis_name="s")

    @functools.partial(
        pl.kernel,
        out_type=jax.ShapeDtypeStruct((BATCH,), jnp.float32),
        mesh=mesh,
        scratch_types=[
            pltpu.VMEM((BPW,), jnp.int32),
            pltpu.VMEM((BPW,), jnp.int32),
            pltpu.VMEM((NPARAM,), jnp.float32),
            pltpu.VMEM((CHUNK, EMB_DIM), jnp.float32),
            pltpu.VMEM((CHUNK, EMB_DIM), jnp.float32),
            pltpu.VMEM((BPW,), jnp.float32),
            pltpu.SemaphoreType.DMA,
            pltpu.SemaphoreType.DMA,
        ],
        compiler_params=pltpu.CompilerParams(needs_layout_passes=False),
    )
    def k(idx0_hbm, idx1_hbm, t0_hbm, t1_hbm, p_hbm, out_hbm,
          idx0_v, idx1_v, p_v, r0_v, r1_v, out_v, sem0, sem1):
        wid = lax.axis_index("s") * NC + lax.axis_index("c")
        base = wid * BPW
        pltpu.sync_copy(idx0_hbm.at[pl.ds(base, BPW)], idx0_v)
        pltpu.sync_copy(idx1_hbm.at[pl.ds(base, BPW)], idx1_v)
        pltpu.sync_copy(p_hbm, p_v)

        lane = lax.iota(jnp.int32, L)

        def chunk_body(c, carry):
            for idx_v, t_hbm, r_v, sem in ((idx0_v, t0_hbm, r0_v, sem0),
                                           (idx1_v, t1_hbm, r1_v, sem1)):
                for g in range(CHUNK // L):
                    iv = idx_v[pl.ds(c * CHUNK + g * L, L)]
                    for j in range(L):
                        pltpu.async_copy(
                            t_hbm.at[pl.ds(iv[j], 1)],
                            r_v.at[pl.ds(g * L + j, 1)], sem)
            pltpu.make_async_copy(t0_hbm.at[pl.ds(0, CHUNK)], r0_v,
                                  sem0).wait()
            pltpu.make_async_copy(t1_hbm.at[pl.ds(0, CHUNK)], r1_v,
                                  sem1).wait()

            ngrp = CHUNK // L
            b1h = [_splat(p_v, 256 + h) for h in range(4)]
            acc = [[b1h[h] for h in range(4)] for _ in range(ngrp)]
            for rows_v, w_off in ((r0_v, 0), (r1_v, EMB_DIM)):
                for f in range(EMB_DIM):
                    w = [_splat(p_v, h * 64 + w_off + f) for h in range(4)]
                    col = jnp.full((L,), f, jnp.int32)
                    for g in range(ngrp):
                        v = plsc.load_gather(rows_v, [lane + g * L, col])
                        for h in range(4):
                            acc[g][h] = acc[g][h] + v * w[h]
            w2h = [_splat(p_v, 260 + h) for h in range(4)]
            b2v = _splat(p_v, 264)
            for g in range(ngrp):
                out16 = b2v
                for h in range(4):
                    z = jnp.maximum(acc[g][h], 0.0)
                    out16 = out16 + z * w2h[h]
                out_v[pl.ds(c * CHUNK + g * L, L)] = out16
            return carry

        lax.fori_loop(0, NCHUNK, chunk_body, None)
        pltpu.sync_copy(out_v, out_hbm.at[pl.ds(base, BPW)])

    return k(idx0, idx1, table0, table1, params)


def kernel(indices, table0, table1, W1, b1, W2, b2):
    idx0 = indices[0].astype(jnp.int32)
    idx1 = indices[1].astype(jnp.int32)
    params = jnp.concatenate(
        [W1.ravel(), b1.ravel(), W2.ravel(), b2.ravel(),
         jnp.zeros((NPARAM - 265,), jnp.float32)])
    out = _fused_dlrm(idx0, idx1, table0, table1, params)
    return out.reshape(BATCH, 1)
